# baseline (speedup 1.0000x reference)
import functools

import jax
import jax.numpy as jnp
from jax import lax
from jax.experimental import pallas as pl
from jax.experimental.pallas import tpu as pltpu
from jax.experimental.pallas import tpu_sc as plsc

EMB_DIM = 64
BATCH = 16384
NEG = 5

NUM_CORES = 2
NUM_SUBCORES = 16
NUM_WORKERS = NUM_CORES * NUM_SUBCORES
ROWS_PER_WORKER = BATCH // NUM_WORKERS
CHUNK = 128
NCHUNKS = ROWS_PER_WORKER // CHUNK
LANES = 16


def _sc_dots_kernel(pos_u_hbm, pos_v_hbm, negf_hbm, uw_hbm, vw_hbm,
                    pos_out, neg_out,
                    idxu, idxv, idxn, idx2u, idx2v, idx2n,
                    urows, vrows, nrows, pdots, ndots,
                    sem):
    wid = lax.axis_index("s") * NUM_CORES + lax.axis_index("c")
    iota = lax.iota(jnp.int32, LANES)

    def chunk_body(chunk, _):
        base = wid * ROWS_PER_WORKER + chunk * CHUNK

        pltpu.sync_copy(pos_u_hbm.at[pl.ds(base, CHUNK)], idxu)
        pltpu.sync_copy(pos_v_hbm.at[pl.ds(base, CHUNK)], idxv)
        for j in range(NEG):
            pltpu.sync_copy(
                negf_hbm.at[pl.ds(base * NEG + j * CHUNK, CHUNK)],
                idxn.at[j])

        def srow(x):
            return ((x >> 9) << 8) + (x & 255)

        def halve(g, _):
            sl = pl.ds(g * LANES, LANES)
            idx2u[sl] = srow(idxu[sl])
            idx2v[sl] = srow(idxv[sl])
            for j in range(NEG):
                sl2 = pl.ds(j * CHUNK + g * LANES, LANES)
                idx2n[sl2] = srow(idxn[j, sl])
            return 0

        lax.fori_loop(0, CHUNK // LANES, halve, 0)

        cps = [pltpu.async_copy(uw_hbm.at[idx2u], urows, sem),
               pltpu.async_copy(vw_hbm.at[idx2v], vrows, sem)]
        for j in range(NEG):
            cps.append(pltpu.async_copy(
                vw_hbm.at[idx2n.at[pl.ds(j * CHUNK, CHUNK)]],
                nrows.at[pl.ds(j * CHUNK, CHUNK)], sem))
        for cp in cps:
            cp.wait()

        def group_body(g, _):
            r0 = g * LANES
            row = r0 + iota
            sl = pl.ds(r0, LANES)
            hu = ((idxu[sl] >> 8) & 1) * EMB_DIM
            hv = ((idxv[sl] >> 8) & 1) * EMB_DIM
            hn = [((idxn[j, sl] >> 8) & 1) * EMB_DIM for j in range(NEG)]
            nrow = [row + j * CHUNK for j in range(NEG)]
            acc_p = jnp.zeros((LANES,), jnp.float32)
            acc_n = [jnp.zeros((LANES,), jnp.float32) for _ in range(NEG)]
            for c in range(EMB_DIM):
                uc = plsc.load_gather(urows, [row, hu + c])
                vc = plsc.load_gather(vrows, [row, hv + c])
                acc_p = acc_p + uc * vc
                for j in range(NEG):
                    nc = plsc.load_gather(nrows, [nrow[j], hn[j] + c])
                    acc_n[j] = acc_n[j] + uc * nc
            pdots[sl] = acc_p
            for j in range(NEG):
                ndots[pl.ds(j * CHUNK + r0, LANES)] = acc_n[j]
            return 0

        lax.fori_loop(0, CHUNK // LANES, group_body, 0)

        pltpu.sync_copy(pdots, pos_out.at[pl.ds(base, CHUNK)])
        pltpu.sync_copy(ndots, neg_out.at[pl.ds(base * NEG, CHUNK * NEG)])
        return 0

    lax.fori_loop(0, NCHUNKS, chunk_body, 0)


_sc_dots = functools.partial(
    pl.kernel,
    mesh=plsc.VectorSubcoreMesh(core_axis_name="c", subcore_axis_name="s"),
    out_type=[jax.ShapeDtypeStruct((BATCH,), jnp.float32),
              jax.ShapeDtypeStruct((BATCH * NEG,), jnp.float32)],
    scratch_types=[
        pltpu.VMEM((CHUNK,), jnp.int32),
        pltpu.VMEM((CHUNK,), jnp.int32),
        pltpu.VMEM((NEG, CHUNK), jnp.int32),
        pltpu.VMEM((CHUNK,), jnp.int32),
        pltpu.VMEM((CHUNK,), jnp.int32),
        pltpu.VMEM((NEG * CHUNK,), jnp.int32),
        pltpu.VMEM((CHUNK, 2 * EMB_DIM), jnp.float32),
        pltpu.VMEM((CHUNK, 2 * EMB_DIM), jnp.float32),
        pltpu.VMEM((CHUNK * NEG, 2 * EMB_DIM), jnp.float32),
        pltpu.VMEM((CHUNK,), jnp.float32),
        pltpu.VMEM((CHUNK * NEG,), jnp.float32),
        pltpu.SemaphoreType.DMA,
    ],
    compiler_params=pltpu.CompilerParams(needs_layout_passes=False),
)(_sc_dots_kernel)


def _reduce_body(p_ref, n_ref, o_ref):
    s = jnp.sum(jax.nn.log_sigmoid(p_ref[...]))
    s = s + jnp.sum(jax.nn.log_sigmoid(-n_ref[...]))
    o_ref[...] = jnp.broadcast_to(-s, (1, 1))


_RP_W = 512
_RP_GRID = (1000000 + _RP_W - 1) // _RP_W
_N2 = _RP_GRID * (_RP_W // 2)


def _repack_body(t_ref, o_ref):
    x = t_ref[...]
    y = jnp.concatenate(
        [x[:, : _RP_W // 2], x[:, _RP_W // 2:]], axis=0)
    eye = jnp.eye(2 * EMB_DIM, dtype=jnp.float32)
    o_ref[...] = jax.lax.dot_general(
        y, eye, (((0,), (0,)), ((), ())),
        preferred_element_type=jnp.float32)


_repack = pl.pallas_call(
    _repack_body,
    grid=(_RP_GRID,),
    in_specs=[pl.BlockSpec((EMB_DIM, _RP_W), lambda j: (0, j))],
    out_specs=pl.BlockSpec((_RP_W // 2, 128), lambda j: (j, 0)),
    out_shape=jax.ShapeDtypeStruct((_N2, 128), jnp.float32),
)


def kernel(pos_u, pos_v, neg_v, u_weight, v_weight):
    pos_u = pos_u.astype(jnp.int32)
    pos_v = pos_v.astype(jnp.int32)
    neg_flat = neg_v.astype(jnp.int32).reshape(BATCH * NEG)

    u2 = _repack(u_weight.T)
    v2 = _repack(v_weight.T)

    pos_dots, neg_dots = _sc_dots(pos_u, pos_v, neg_flat, u2, v2)

    out = pl.pallas_call(
        _reduce_body,
        out_shape=jax.ShapeDtypeStruct((1, 1), jnp.float32),
    )(pos_dots.reshape(BATCH // 128, 128),
      neg_dots.reshape(BATCH * NEG // 128, 128))
    return out[0, 0]


# speedup vs baseline: 3.6221x; 3.6221x over previous
import functools

import jax
import jax.numpy as jnp
from jax import lax
from jax.experimental import pallas as pl
from jax.experimental.pallas import tpu as pltpu
from jax.experimental.pallas import tpu_sc as plsc

EMB_DIM = 64
BATCH = 16384
NEG = 5

NUM_CORES = 2
NUM_SUBCORES = 16
NUM_WORKERS = NUM_CORES * NUM_SUBCORES
ROWS_PER_WORKER = BATCH // NUM_WORKERS
CHUNK = 128
NCHUNKS = ROWS_PER_WORKER // CHUNK
LANES = 16


def _sc_dots_kernel(pos_u_hbm, pos_v_hbm, negf_hbm, uw_hbm, vw_hbm,
                    pos_out, neg_out,
                    idxu, idxv, idxn, idx2u, idx2v, idx2n,
                    urows, vrows, nrows, pdots, ndots,
                    sem):
    wid = lax.axis_index("s") * NUM_CORES + lax.axis_index("c")
    iota = lax.iota(jnp.int32, LANES)

    def chunk_body(chunk, _):
        base = wid * ROWS_PER_WORKER + chunk * CHUNK

        pltpu.sync_copy(pos_u_hbm.at[pl.ds(base, CHUNK)], idxu)
        pltpu.sync_copy(pos_v_hbm.at[pl.ds(base, CHUNK)], idxv)
        for j in range(NEG):
            pltpu.sync_copy(
                negf_hbm.at[pl.ds(base * NEG + j * CHUNK, CHUNK)],
                idxn.at[j])

        def srow(x):
            return ((x >> 9) << 8) + (x & 255)

        def halve(g, _):
            sl = pl.ds(g * LANES, LANES)
            idx2u[sl] = srow(idxu[sl])
            idx2v[sl] = srow(idxv[sl])
            for j in range(NEG):
                sl2 = pl.ds(j * CHUNK + g * LANES, LANES)
                idx2n[sl2] = srow(idxn[j, sl])
            return 0

        lax.fori_loop(0, CHUNK // LANES, halve, 0)

        cps = [pltpu.async_copy(uw_hbm.at[idx2u], urows, sem),
               pltpu.async_copy(vw_hbm.at[idx2v], vrows, sem)]
        for j in range(NEG):
            cps.append(pltpu.async_copy(
                vw_hbm.at[idx2n.at[pl.ds(j * CHUNK, CHUNK)]],
                nrows.at[pl.ds(j * CHUNK, CHUNK)], sem))
        for cp in cps:
            cp.wait()

        def group_body(g, _):
            r0 = g * LANES
            row = r0 + iota
            sl = pl.ds(r0, LANES)
            hu = ((idxu[sl] >> 8) & 1) * EMB_DIM
            hv = ((idxv[sl] >> 8) & 1) * EMB_DIM
            hn = [((idxn[j, sl] >> 8) & 1) * EMB_DIM for j in range(NEG)]
            nrow = [row + j * CHUNK for j in range(NEG)]
            acc_p = jnp.zeros((LANES,), jnp.float32)
            acc_n = [jnp.zeros((LANES,), jnp.float32) for _ in range(NEG)]
            for c in range(EMB_DIM):
                uc = plsc.load_gather(urows, [row, hu + c])
                vc = plsc.load_gather(vrows, [row, hv + c])
                acc_p = acc_p + uc * vc
                for j in range(NEG):
                    nc = plsc.load_gather(nrows, [nrow[j], hn[j] + c])
                    acc_n[j] = acc_n[j] + uc * nc
            pdots[sl] = acc_p
            for j in range(NEG):
                ndots[pl.ds(j * CHUNK + r0, LANES)] = acc_n[j]
            return 0

        lax.fori_loop(0, CHUNK // LANES, group_body, 0)

        pltpu.sync_copy(pdots, pos_out.at[pl.ds(base, CHUNK)])
        pltpu.sync_copy(ndots, neg_out.at[pl.ds(base * NEG, CHUNK * NEG)])
        return 0

    lax.fori_loop(0, NCHUNKS, chunk_body, 0)


_sc_dots = functools.partial(
    pl.kernel,
    mesh=plsc.VectorSubcoreMesh(core_axis_name="c", subcore_axis_name="s"),
    out_type=[jax.ShapeDtypeStruct((BATCH,), jnp.float32),
              jax.ShapeDtypeStruct((BATCH * NEG,), jnp.float32)],
    scratch_types=[
        pltpu.VMEM((CHUNK,), jnp.int32),
        pltpu.VMEM((CHUNK,), jnp.int32),
        pltpu.VMEM((NEG, CHUNK), jnp.int32),
        pltpu.VMEM((CHUNK,), jnp.int32),
        pltpu.VMEM((CHUNK,), jnp.int32),
        pltpu.VMEM((NEG * CHUNK,), jnp.int32),
        pltpu.VMEM((CHUNK, 2 * EMB_DIM), jnp.float32),
        pltpu.VMEM((CHUNK, 2 * EMB_DIM), jnp.float32),
        pltpu.VMEM((CHUNK * NEG, 2 * EMB_DIM), jnp.float32),
        pltpu.VMEM((CHUNK,), jnp.float32),
        pltpu.VMEM((CHUNK * NEG,), jnp.float32),
        pltpu.SemaphoreType.DMA,
    ],
    compiler_params=pltpu.CompilerParams(needs_layout_passes=False),
)(_sc_dots_kernel)


def _reduce_body(p_ref, n_ref, o_ref):
    s = jnp.sum(jax.nn.log_sigmoid(p_ref[...]))
    s = s + jnp.sum(jax.nn.log_sigmoid(-n_ref[...]))
    o_ref[...] = jnp.broadcast_to(-s, (1, 1))


_RP_W = 4096
_RP_GRID = (1000000 + _RP_W - 1) // _RP_W
_N2 = _RP_GRID * (_RP_W // 2)


def _repack_body(t_ref, o_ref):
    x = t_ref[...]
    y = jnp.concatenate(
        [x[:, : _RP_W // 2], x[:, _RP_W // 2:]], axis=0)
    eye = jnp.eye(2 * EMB_DIM, dtype=jnp.float32)
    o_ref[...] = jax.lax.dot_general(
        y, eye, (((0,), (0,)), ((), ())),
        preferred_element_type=jnp.float32)


_repack = pl.pallas_call(
    _repack_body,
    grid=(_RP_GRID,),
    in_specs=[pl.BlockSpec((EMB_DIM, _RP_W), lambda j: (0, j))],
    out_specs=pl.BlockSpec((_RP_W // 2, 128), lambda j: (j, 0)),
    out_shape=jax.ShapeDtypeStruct((_N2, 128), jnp.float32),
)


def kernel(pos_u, pos_v, neg_v, u_weight, v_weight):
    pos_u = pos_u.astype(jnp.int32)
    pos_v = pos_v.astype(jnp.int32)
    neg_flat = neg_v.astype(jnp.int32).reshape(BATCH * NEG)

    u2 = _repack(u_weight.T)
    v2 = _repack(v_weight.T)

    pos_dots, neg_dots = _sc_dots(pos_u, pos_v, neg_flat, u2, v2)

    out = pl.pallas_call(
        _reduce_body,
        out_shape=jax.ShapeDtypeStruct((1, 1), jnp.float32),
    )(pos_dots.reshape(BATCH // 128, 128),
      neg_dots.reshape(BATCH * NEG // 128, 128))
    return out[0, 0]


# speedup vs baseline: 4.6231x; 1.2764x over previous
import functools

import jax
import jax.numpy as jnp
from jax import lax
from jax.experimental import pallas as pl
from jax.experimental.pallas import tpu as pltpu
from jax.experimental.pallas import tpu_sc as plsc

EMB_DIM = 64
BATCH = 16384
NEG = 5

NUM_CORES = 2
NUM_SUBCORES = 16
NUM_WORKERS = NUM_CORES * NUM_SUBCORES
ROWS_PER_WORKER = BATCH // NUM_WORKERS
CHUNK = 128
NCHUNKS = ROWS_PER_WORKER // CHUNK
LANES = 16


def _sc_dots_kernel(pos_u_hbm, pos_v_hbm, negf_hbm, uw_hbm, vw_hbm,
                    pos_out, neg_out,
                    idxu, idxv, idxn, idx2u, idx2v, idx2n,
                    urows, vrows, nrows, pdots, ndots,
                    sem):
    wid = lax.axis_index("s") * NUM_CORES + lax.axis_index("c")
    iota = lax.iota(jnp.int32, LANES)

    def chunk_body(chunk, _):
        base = wid * ROWS_PER_WORKER + chunk * CHUNK

        pltpu.sync_copy(pos_u_hbm.at[pl.ds(base, CHUNK)], idxu)
        pltpu.sync_copy(pos_v_hbm.at[pl.ds(base, CHUNK)], idxv)
        for j in range(NEG):
            pltpu.sync_copy(
                negf_hbm.at[pl.ds(base * NEG + j * CHUNK, CHUNK)],
                idxn.at[j])

        def srow(x):
            return ((x >> 9) << 8) + (x & 255)

        def halve(g, _):
            sl = pl.ds(g * LANES, LANES)
            idx2u[sl] = srow(idxu[sl])
            idx2v[sl] = srow(idxv[sl])
            for j in range(NEG):
                sl2 = pl.ds(j * CHUNK + g * LANES, LANES)
                idx2n[sl2] = srow(idxn[j, sl])
            return 0

        lax.fori_loop(0, CHUNK // LANES, halve, 0)

        cps = [pltpu.async_copy(uw_hbm.at[idx2u], urows, sem),
               pltpu.async_copy(vw_hbm.at[idx2v], vrows, sem)]
        for j in range(NEG):
            cps.append(pltpu.async_copy(
                vw_hbm.at[idx2n.at[pl.ds(j * CHUNK, CHUNK)]],
                nrows.at[pl.ds(j * CHUNK, CHUNK)], sem))
        for cp in cps:
            cp.wait()

        def group_body(g, _):
            r0 = g * LANES
            row = r0 + iota
            sl = pl.ds(r0, LANES)
            hu = ((idxu[sl] >> 8) & 1) * EMB_DIM
            hv = ((idxv[sl] >> 8) & 1) * EMB_DIM
            hn = [((idxn[j, sl] >> 8) & 1) * EMB_DIM for j in range(NEG)]
            nrow = [row + j * CHUNK for j in range(NEG)]
            acc_p = jnp.zeros((LANES,), jnp.float32)
            acc_n = [jnp.zeros((LANES,), jnp.float32) for _ in range(NEG)]
            for c in range(EMB_DIM):
                uc = plsc.load_gather(urows, [row, hu + c])
                vc = plsc.load_gather(vrows, [row, hv + c])
                acc_p = acc_p + uc * vc
                for j in range(NEG):
                    nc = plsc.load_gather(nrows, [nrow[j], hn[j] + c])
                    acc_n[j] = acc_n[j] + uc * nc
            pdots[sl] = acc_p
            for j in range(NEG):
                ndots[pl.ds(j * CHUNK + r0, LANES)] = acc_n[j]
            return 0

        lax.fori_loop(0, CHUNK // LANES, group_body, 0)

        pltpu.sync_copy(pdots, pos_out.at[pl.ds(base, CHUNK)])
        pltpu.sync_copy(ndots, neg_out.at[pl.ds(base * NEG, CHUNK * NEG)])
        return 0

    lax.fori_loop(0, NCHUNKS, chunk_body, 0)


_sc_dots = functools.partial(
    pl.kernel,
    mesh=plsc.VectorSubcoreMesh(core_axis_name="c", subcore_axis_name="s"),
    out_type=[jax.ShapeDtypeStruct((BATCH,), jnp.float32),
              jax.ShapeDtypeStruct((BATCH * NEG,), jnp.float32)],
    scratch_types=[
        pltpu.VMEM((CHUNK,), jnp.int32),
        pltpu.VMEM((CHUNK,), jnp.int32),
        pltpu.VMEM((NEG, CHUNK), jnp.int32),
        pltpu.VMEM((CHUNK,), jnp.int32),
        pltpu.VMEM((CHUNK,), jnp.int32),
        pltpu.VMEM((NEG * CHUNK,), jnp.int32),
        pltpu.VMEM((CHUNK, 2 * EMB_DIM), jnp.float32),
        pltpu.VMEM((CHUNK, 2 * EMB_DIM), jnp.float32),
        pltpu.VMEM((CHUNK * NEG, 2 * EMB_DIM), jnp.float32),
        pltpu.VMEM((CHUNK,), jnp.float32),
        pltpu.VMEM((CHUNK * NEG,), jnp.float32),
        pltpu.SemaphoreType.DMA,
    ],
    compiler_params=pltpu.CompilerParams(needs_layout_passes=False),
)(_sc_dots_kernel)


def _reduce_body(p_ref, n_ref, o_ref):
    s = jnp.sum(jax.nn.log_sigmoid(p_ref[...]))
    s = s + jnp.sum(jax.nn.log_sigmoid(-n_ref[...]))
    o_ref[...] = jnp.broadcast_to(-s, (1, 1))


_RP_W = 8192
_RP_GRID = (1000000 + _RP_W - 1) // _RP_W
_N2 = _RP_GRID * (_RP_W // 2)


def _repack_body(t_ref, o_ref):
    x = t_ref[...]
    y = jnp.concatenate(
        [x[:, : _RP_W // 2], x[:, _RP_W // 2:]], axis=0)
    eye = jnp.eye(2 * EMB_DIM, dtype=jnp.float32)
    o_ref[...] = jax.lax.dot_general(
        y, eye, (((0,), (0,)), ((), ())),
        preferred_element_type=jnp.float32)


_repack = pl.pallas_call(
    _repack_body,
    grid=(_RP_GRID,),
    in_specs=[pl.BlockSpec((EMB_DIM, _RP_W), lambda j: (0, j))],
    out_specs=pl.BlockSpec((_RP_W // 2, 128), lambda j: (j, 0)),
    out_shape=jax.ShapeDtypeStruct((_N2, 128), jnp.float32),
)


def kernel(pos_u, pos_v, neg_v, u_weight, v_weight):
    pos_u = pos_u.astype(jnp.int32)
    pos_v = pos_v.astype(jnp.int32)
    neg_flat = neg_v.astype(jnp.int32).reshape(BATCH * NEG)

    u2 = _repack(u_weight.T)
    v2 = _repack(v_weight.T)

    pos_dots, neg_dots = _sc_dots(pos_u, pos_v, neg_flat, u2, v2)

    out = pl.pallas_call(
        _reduce_body,
        out_shape=jax.ShapeDtypeStruct((1, 1), jnp.float32),
    )(pos_dots.reshape(BATCH // 128, 128),
      neg_dots.reshape(BATCH * NEG // 128, 128))
    return out[0, 0]


# speedup vs baseline: 5.1260x; 1.1088x over previous
import functools

import jax
import jax.numpy as jnp
from jax import lax
from jax.experimental import pallas as pl
from jax.experimental.pallas import tpu as pltpu
from jax.experimental.pallas import tpu_sc as plsc

EMB_DIM = 64
BATCH = 16384
NEG = 5

NUM_CORES = 2
NUM_SUBCORES = 16
NUM_WORKERS = NUM_CORES * NUM_SUBCORES
ROWS_PER_WORKER = BATCH // NUM_WORKERS
CHUNK = 128
NCHUNKS = ROWS_PER_WORKER // CHUNK
LANES = 16


def _sc_dots_kernel(pos_u_hbm, pos_v_hbm, negf_hbm, uw_hbm, vw_hbm,
                    pos_out, neg_out,
                    idxu, idxv, idxn, idx2u, idx2v, idx2n,
                    urows, vrows, nrows, pdots, ndots,
                    sem):
    wid = lax.axis_index("s") * NUM_CORES + lax.axis_index("c")
    iota = lax.iota(jnp.int32, LANES)

    def chunk_body(chunk, _):
        base = wid * ROWS_PER_WORKER + chunk * CHUNK

        pltpu.sync_copy(pos_u_hbm.at[pl.ds(base, CHUNK)], idxu)
        pltpu.sync_copy(pos_v_hbm.at[pl.ds(base, CHUNK)], idxv)
        for j in range(NEG):
            pltpu.sync_copy(
                negf_hbm.at[pl.ds(base * NEG + j * CHUNK, CHUNK)],
                idxn.at[j])

        def srow(x):
            return ((x >> 9) << 8) + (x & 255)

        def halve(g, _):
            sl = pl.ds(g * LANES, LANES)
            idx2u[sl] = srow(idxu[sl])
            idx2v[sl] = srow(idxv[sl])
            for j in range(NEG):
                sl2 = pl.ds(j * CHUNK + g * LANES, LANES)
                idx2n[sl2] = srow(idxn[j, sl])
            return 0

        lax.fori_loop(0, CHUNK // LANES, halve, 0)

        cps = [pltpu.async_copy(uw_hbm.at[idx2u], urows, sem),
               pltpu.async_copy(vw_hbm.at[idx2v], vrows, sem)]
        for j in range(NEG):
            cps.append(pltpu.async_copy(
                vw_hbm.at[idx2n.at[pl.ds(j * CHUNK, CHUNK)]],
                nrows.at[pl.ds(j * CHUNK, CHUNK)], sem))
        for cp in cps:
            cp.wait()

        def group_body(g, _):
            r0 = g * LANES
            row = r0 + iota
            sl = pl.ds(r0, LANES)
            hu = ((idxu[sl] >> 8) & 1) * EMB_DIM
            hv = ((idxv[sl] >> 8) & 1) * EMB_DIM
            hn = [((idxn[j, sl] >> 8) & 1) * EMB_DIM for j in range(NEG)]
            nrow = [row + j * CHUNK for j in range(NEG)]
            acc_p = jnp.zeros((LANES,), jnp.float32)
            acc_n = [jnp.zeros((LANES,), jnp.float32) for _ in range(NEG)]
            for c in range(EMB_DIM):
                uc = plsc.load_gather(urows, [row, hu + c])
                vc = plsc.load_gather(vrows, [row, hv + c])
                acc_p = acc_p + uc * vc
                for j in range(NEG):
                    nc = plsc.load_gather(nrows, [nrow[j], hn[j] + c])
                    acc_n[j] = acc_n[j] + uc * nc
            pdots[sl] = acc_p
            for j in range(NEG):
                ndots[pl.ds(j * CHUNK + r0, LANES)] = acc_n[j]
            return 0

        lax.fori_loop(0, CHUNK // LANES, group_body, 0)

        pltpu.sync_copy(pdots, pos_out.at[pl.ds(base, CHUNK)])
        pltpu.sync_copy(ndots, neg_out.at[pl.ds(base * NEG, CHUNK * NEG)])
        return 0

    lax.fori_loop(0, NCHUNKS, chunk_body, 0)


_sc_dots = functools.partial(
    pl.kernel,
    mesh=plsc.VectorSubcoreMesh(core_axis_name="c", subcore_axis_name="s"),
    out_type=[jax.ShapeDtypeStruct((BATCH,), jnp.float32),
              jax.ShapeDtypeStruct((BATCH * NEG,), jnp.float32)],
    scratch_types=[
        pltpu.VMEM((CHUNK,), jnp.int32),
        pltpu.VMEM((CHUNK,), jnp.int32),
        pltpu.VMEM((NEG, CHUNK), jnp.int32),
        pltpu.VMEM((CHUNK,), jnp.int32),
        pltpu.VMEM((CHUNK,), jnp.int32),
        pltpu.VMEM((NEG * CHUNK,), jnp.int32),
        pltpu.VMEM((CHUNK, 2 * EMB_DIM), jnp.float32),
        pltpu.VMEM((CHUNK, 2 * EMB_DIM), jnp.float32),
        pltpu.VMEM((CHUNK * NEG, 2 * EMB_DIM), jnp.float32),
        pltpu.VMEM((CHUNK,), jnp.float32),
        pltpu.VMEM((CHUNK * NEG,), jnp.float32),
        pltpu.SemaphoreType.DMA,
    ],
    compiler_params=pltpu.CompilerParams(needs_layout_passes=False),
)(_sc_dots_kernel)


def _reduce_body(p_ref, n_ref, o_ref):
    s = jnp.sum(jax.nn.log_sigmoid(p_ref[...]))
    s = s + jnp.sum(jax.nn.log_sigmoid(-n_ref[...]))
    o_ref[...] = jnp.broadcast_to(-s, (1, 1))


_RP_W = 16384
_RP_GRID = (1000000 + _RP_W - 1) // _RP_W
_N2 = _RP_GRID * (_RP_W // 2)


def _repack_body(t_ref, o_ref):
    x = t_ref[...]
    y = jnp.concatenate(
        [x[:, : _RP_W // 2], x[:, _RP_W // 2:]], axis=0)
    eye = jnp.eye(2 * EMB_DIM, dtype=jnp.float32)
    o_ref[...] = jax.lax.dot_general(
        y, eye, (((0,), (0,)), ((), ())),
        preferred_element_type=jnp.float32)


_repack = pl.pallas_call(
    _repack_body,
    grid=(_RP_GRID,),
    in_specs=[pl.BlockSpec((EMB_DIM, _RP_W), lambda j: (0, j))],
    out_specs=pl.BlockSpec((_RP_W // 2, 128), lambda j: (j, 0)),
    out_shape=jax.ShapeDtypeStruct((_N2, 128), jnp.float32),
)


def kernel(pos_u, pos_v, neg_v, u_weight, v_weight):
    pos_u = pos_u.astype(jnp.int32)
    pos_v = pos_v.astype(jnp.int32)
    neg_flat = neg_v.astype(jnp.int32).reshape(BATCH * NEG)

    u2 = _repack(u_weight.T)
    v2 = _repack(v_weight.T)

    pos_dots, neg_dots = _sc_dots(pos_u, pos_v, neg_flat, u2, v2)

    out = pl.pallas_call(
        _reduce_body,
        out_shape=jax.ShapeDtypeStruct((1, 1), jnp.float32),
    )(pos_dots.reshape(BATCH // 128, 128),
      neg_dots.reshape(BATCH * NEG // 128, 128))
    return out[0, 0]


# speedup vs baseline: 5.3452x; 1.0428x over previous
import functools

import jax
import jax.numpy as jnp
from jax import lax
from jax.experimental import pallas as pl
from jax.experimental.pallas import tpu as pltpu
from jax.experimental.pallas import tpu_sc as plsc

EMB_DIM = 64
BATCH = 16384
NEG = 5

_RP_W = 16384
_RP_LOGW = _RP_W.bit_length() - 1

NUM_CORES = 2
NUM_SUBCORES = 16
NUM_WORKERS = NUM_CORES * NUM_SUBCORES
ROWS_PER_WORKER = BATCH // NUM_WORKERS
CHUNK = 128
NCHUNKS = ROWS_PER_WORKER // CHUNK
LANES = 16


def _sc_dots_kernel(pos_u_hbm, pos_v_hbm, negT_hbm, uw_hbm, vw_hbm,
                    pos_out, neg_out,
                    idxu, idxv, idxn, idx2u, idx2v, idx2n,
                    urows, vrows, nrows, pdots, ndots,
                    sem):
    wid = lax.axis_index("s") * NUM_CORES + lax.axis_index("c")
    iota = lax.iota(jnp.int32, LANES)

    def chunk_body(chunk, _):
        base = wid * ROWS_PER_WORKER + chunk * CHUNK

        pltpu.sync_copy(pos_u_hbm.at[pl.ds(base, CHUNK)], idxu)
        pltpu.sync_copy(pos_v_hbm.at[pl.ds(base, CHUNK)], idxv)
        pltpu.sync_copy(negT_hbm.at[:, pl.ds(base, CHUNK)], idxn)

        def srow(x):
            return ((x >> _RP_LOGW) << (_RP_LOGW - 1)) + (x & (_RP_W // 2 - 1))

        def halve(g, _):
            sl = pl.ds(g * LANES, LANES)
            idx2u[sl] = srow(idxu[sl])
            idx2v[sl] = srow(idxv[sl])
            for j in range(NEG):
                sl2 = pl.ds(j * CHUNK + g * LANES, LANES)
                idx2n[sl2] = srow(idxn[j, sl])
            return 0

        lax.fori_loop(0, CHUNK // LANES, halve, 0)

        cps = [pltpu.async_copy(uw_hbm.at[idx2u], urows, sem),
               pltpu.async_copy(vw_hbm.at[idx2v], vrows, sem)]
        for j in range(NEG):
            cps.append(pltpu.async_copy(
                vw_hbm.at[idx2n.at[pl.ds(j * CHUNK, CHUNK)]],
                nrows.at[pl.ds(j * CHUNK, CHUNK)], sem))
        for cp in cps:
            cp.wait()

        def group_body(g, _):
            r0 = g * LANES
            row = r0 + iota
            sl = pl.ds(r0, LANES)
            hu = ((idxu[sl] >> (_RP_LOGW - 1)) & 1) * EMB_DIM
            hv = ((idxv[sl] >> (_RP_LOGW - 1)) & 1) * EMB_DIM
            hn = [((idxn[j, sl] >> (_RP_LOGW - 1)) & 1) * EMB_DIM
                  for j in range(NEG)]
            nrow = [row + j * CHUNK for j in range(NEG)]
            acc_p = jnp.zeros((LANES,), jnp.float32)
            acc_n = [jnp.zeros((LANES,), jnp.float32) for _ in range(NEG)]
            for c in range(EMB_DIM):
                uc = plsc.load_gather(urows, [row, hu + c])
                vc = plsc.load_gather(vrows, [row, hv + c])
                acc_p = acc_p + uc * vc
                for j in range(NEG):
                    nc = plsc.load_gather(nrows, [nrow[j], hn[j] + c])
                    acc_n[j] = acc_n[j] + uc * nc
            pdots[sl] = acc_p
            for j in range(NEG):
                ndots[pl.ds(j * CHUNK + r0, LANES)] = acc_n[j]
            return 0

        lax.fori_loop(0, CHUNK // LANES, group_body, 0)

        pltpu.sync_copy(pdots, pos_out.at[pl.ds(base, CHUNK)])
        pltpu.sync_copy(ndots, neg_out.at[pl.ds(base * NEG, CHUNK * NEG)])
        return 0

    lax.fori_loop(0, NCHUNKS, chunk_body, 0)


_sc_dots = functools.partial(
    pl.kernel,
    mesh=plsc.VectorSubcoreMesh(core_axis_name="c", subcore_axis_name="s"),
    out_type=[jax.ShapeDtypeStruct((BATCH,), jnp.float32),
              jax.ShapeDtypeStruct((BATCH * NEG,), jnp.float32)],
    scratch_types=[
        pltpu.VMEM((CHUNK,), jnp.int32),
        pltpu.VMEM((CHUNK,), jnp.int32),
        pltpu.VMEM((NEG, CHUNK), jnp.int32),
        pltpu.VMEM((CHUNK,), jnp.int32),
        pltpu.VMEM((CHUNK,), jnp.int32),
        pltpu.VMEM((NEG * CHUNK,), jnp.int32),
        pltpu.VMEM((CHUNK, 2 * EMB_DIM), jnp.float32),
        pltpu.VMEM((CHUNK, 2 * EMB_DIM), jnp.float32),
        pltpu.VMEM((CHUNK * NEG, 2 * EMB_DIM), jnp.float32),
        pltpu.VMEM((CHUNK,), jnp.float32),
        pltpu.VMEM((CHUNK * NEG,), jnp.float32),
        pltpu.SemaphoreType.DMA,
    ],
    compiler_params=pltpu.CompilerParams(needs_layout_passes=False),
)(_sc_dots_kernel)


def _reduce_body(p_ref, n_ref, o_ref):
    s = jnp.sum(jax.nn.log_sigmoid(p_ref[...]))
    s = s + jnp.sum(jax.nn.log_sigmoid(-n_ref[...]))
    o_ref[...] = jnp.broadcast_to(-s, (1, 1))


_RP_GRID = (1000000 + _RP_W - 1) // _RP_W
_N2 = _RP_GRID * (_RP_W // 2)


def _repack_body(t_ref, o_ref):
    x = t_ref[...]
    y = jnp.concatenate(
        [x[:, : _RP_W // 2], x[:, _RP_W // 2:]], axis=0)
    eye = jnp.eye(2 * EMB_DIM, dtype=jnp.float32)
    o_ref[...] = jax.lax.dot_general(
        y, eye, (((0,), (0,)), ((), ())),
        preferred_element_type=jnp.float32)


_repack = pl.pallas_call(
    _repack_body,
    grid=(_RP_GRID,),
    in_specs=[pl.BlockSpec((EMB_DIM, _RP_W), lambda j: (0, j))],
    out_specs=pl.BlockSpec((_RP_W // 2, 128), lambda j: (j, 0)),
    out_shape=jax.ShapeDtypeStruct((_N2, 128), jnp.float32),
)


def kernel(pos_u, pos_v, neg_v, u_weight, v_weight):
    pos_u = pos_u.astype(jnp.int32)
    pos_v = pos_v.astype(jnp.int32)
    neg_t = neg_v.astype(jnp.int32).T

    u2 = _repack(u_weight.T)
    v2 = _repack(v_weight.T)

    pos_dots, neg_dots = _sc_dots(pos_u, pos_v, neg_t, u2, v2)

    out = pl.pallas_call(
        _reduce_body,
        out_shape=jax.ShapeDtypeStruct((1, 1), jnp.float32),
    )(pos_dots.reshape(BATCH // 128, 128),
      neg_dots.reshape(BATCH * NEG // 128, 128))
    return out[0, 0]


# speedup vs baseline: 5.5360x; 1.0357x over previous
import functools

import jax
import jax.numpy as jnp
from jax import lax
from jax.experimental import pallas as pl
from jax.experimental.pallas import tpu as pltpu
from jax.experimental.pallas import tpu_sc as plsc

EMB_DIM = 64
BATCH = 16384
NEG = 5

_RP_W = 16384
_RP_LOGW = _RP_W.bit_length() - 1

NUM_CORES = 2
NUM_SUBCORES = 16
NUM_WORKERS = NUM_CORES * NUM_SUBCORES
ROWS_PER_WORKER = BATCH // NUM_WORKERS
CHUNK = 64
NCHUNKS = ROWS_PER_WORKER // CHUNK
LANES = 16


def _sc_dots_kernel(pos_u_hbm, pos_v_hbm, negT_hbm, uw_hbm, vw_hbm,
                    pos_out, neg_out,
                    idxu, idxv, idxn, idx2u, idx2v, idx2n,
                    urows0, vrows0, nrows0, urows1, vrows1, nrows1,
                    pdots, ndots,
                    sem0, sem1):
    wid = lax.axis_index("s") * NUM_CORES + lax.axis_index("c")
    iota = lax.iota(jnp.int32, LANES)
    wbase = wid * ROWS_PER_WORKER

    pltpu.sync_copy(pos_u_hbm.at[pl.ds(wbase, ROWS_PER_WORKER)], idxu)
    pltpu.sync_copy(pos_v_hbm.at[pl.ds(wbase, ROWS_PER_WORKER)], idxv)
    pltpu.sync_copy(negT_hbm.at[:, pl.ds(wbase, ROWS_PER_WORKER)], idxn)

    def srow(x):
        return ((x >> _RP_LOGW) << (_RP_LOGW - 1)) + (x & (_RP_W // 2 - 1))

    def halve(g, _):
        sl = pl.ds(g * LANES, LANES)
        idx2u[sl] = srow(idxu[sl])
        idx2v[sl] = srow(idxv[sl])
        for j in range(NEG):
            sl2 = pl.ds(j * ROWS_PER_WORKER + g * LANES, LANES)
            idx2n[sl2] = srow(idxn[j, sl])
        return 0

    lax.fori_loop(0, ROWS_PER_WORKER // LANES, halve, 0)

    def fire(chunk, bufs, sem):
        urows, vrows, nrows = bufs
        off = chunk * CHUNK
        pltpu.async_copy(uw_hbm.at[idx2u.at[pl.ds(off, CHUNK)]], urows, sem)
        pltpu.async_copy(vw_hbm.at[idx2v.at[pl.ds(off, CHUNK)]], vrows, sem)
        for j in range(NEG):
            pltpu.async_copy(
                vw_hbm.at[idx2n.at[pl.ds(j * ROWS_PER_WORKER + off, CHUNK)]],
                nrows.at[pl.ds(j * CHUNK, CHUNK)], sem)

    def drain(bufs, sem):
        urows, vrows, nrows = bufs
        pltpu.make_async_copy(uw_hbm.at[pl.ds(0, CHUNK)], urows, sem).wait()
        pltpu.make_async_copy(vw_hbm.at[pl.ds(0, CHUNK)], vrows, sem).wait()
        pltpu.make_async_copy(
            vw_hbm.at[pl.ds(0, NEG * CHUNK)], nrows, sem).wait()

    def compute(chunk, bufs):
        urows, vrows, nrows = bufs
        off = chunk * CHUNK

        def group_body(g, _):
            r0 = g * LANES
            row = r0 + iota
            sl = pl.ds(off + r0, LANES)
            hu = ((idxu[sl] >> (_RP_LOGW - 1)) & 1) * EMB_DIM
            hv = ((idxv[sl] >> (_RP_LOGW - 1)) & 1) * EMB_DIM
            hn = [((idxn[j, sl] >> (_RP_LOGW - 1)) & 1) * EMB_DIM
                  for j in range(NEG)]
            nrow = [row + j * CHUNK for j in range(NEG)]
            acc_p = jnp.zeros((LANES,), jnp.float32)
            acc_n = [jnp.zeros((LANES,), jnp.float32) for _ in range(NEG)]
            for c in range(EMB_DIM):
                uc = plsc.load_gather(urows, [row, hu + c])
                vc = plsc.load_gather(vrows, [row, hv + c])
                acc_p = acc_p + uc * vc
                for j in range(NEG):
                    nc = plsc.load_gather(nrows, [nrow[j], hn[j] + c])
                    acc_n[j] = acc_n[j] + uc * nc
            pdots[pl.ds(r0, LANES)] = acc_p
            for j in range(NEG):
                ndots[pl.ds(j * CHUNK + r0, LANES)] = acc_n[j]
            return 0

        lax.fori_loop(0, CHUNK // LANES, group_body, 0)

        base = wbase + off
        pltpu.sync_copy(pdots, pos_out.at[pl.ds(base, CHUNK)])
        pltpu.sync_copy(ndots, neg_out.at[pl.ds(base * NEG, CHUNK * NEG)])

    bufs0 = (urows0, vrows0, nrows0)
    bufs1 = (urows1, vrows1, nrows1)

    fire(0, bufs0, sem0)

    def chunk_body(chunk, _):
        @pl.when(chunk % 2 == 0)
        def _():
            fire(chunk + 1, bufs1, sem1)
            drain(bufs0, sem0)
            compute(chunk, bufs0)

        @pl.when(chunk % 2 == 1)
        def _():
            fire(chunk + 1, bufs0, sem0)
            drain(bufs1, sem1)
            compute(chunk, bufs1)

        return 0

    lax.fori_loop(0, NCHUNKS - 1, chunk_body, 0)
    drain(bufs1, sem1)
    compute(NCHUNKS - 1, bufs1)


_sc_dots = functools.partial(
    pl.kernel,
    mesh=plsc.VectorSubcoreMesh(core_axis_name="c", subcore_axis_name="s"),
    out_type=[jax.ShapeDtypeStruct((BATCH,), jnp.float32),
              jax.ShapeDtypeStruct((BATCH * NEG,), jnp.float32)],
    scratch_types=[
        pltpu.VMEM((ROWS_PER_WORKER,), jnp.int32),
        pltpu.VMEM((ROWS_PER_WORKER,), jnp.int32),
        pltpu.VMEM((NEG, ROWS_PER_WORKER), jnp.int32),
        pltpu.VMEM((ROWS_PER_WORKER,), jnp.int32),
        pltpu.VMEM((ROWS_PER_WORKER,), jnp.int32),
        pltpu.VMEM((NEG * ROWS_PER_WORKER,), jnp.int32),
        pltpu.VMEM((CHUNK, 2 * EMB_DIM), jnp.float32),
        pltpu.VMEM((CHUNK, 2 * EMB_DIM), jnp.float32),
        pltpu.VMEM((CHUNK * NEG, 2 * EMB_DIM), jnp.float32),
        pltpu.VMEM((CHUNK, 2 * EMB_DIM), jnp.float32),
        pltpu.VMEM((CHUNK, 2 * EMB_DIM), jnp.float32),
        pltpu.VMEM((CHUNK * NEG, 2 * EMB_DIM), jnp.float32),
        pltpu.VMEM((CHUNK,), jnp.float32),
        pltpu.VMEM((CHUNK * NEG,), jnp.float32),
        pltpu.SemaphoreType.DMA,
        pltpu.SemaphoreType.DMA,
    ],
    compiler_params=pltpu.CompilerParams(needs_layout_passes=False),
)(_sc_dots_kernel)


def _reduce_body(p_ref, n_ref, o_ref):
    s = jnp.sum(jax.nn.log_sigmoid(p_ref[...]))
    s = s + jnp.sum(jax.nn.log_sigmoid(-n_ref[...]))
    o_ref[...] = jnp.broadcast_to(-s, (1, 1))


_RP_GRID = (1000000 + _RP_W - 1) // _RP_W
_N2 = _RP_GRID * (_RP_W // 2)


def _repack_body(t_ref, o_ref):
    x = t_ref[...]
    y = jnp.concatenate(
        [x[:, : _RP_W // 2], x[:, _RP_W // 2:]], axis=0)
    eye = jnp.eye(2 * EMB_DIM, dtype=jnp.float32)
    o_ref[...] = jax.lax.dot_general(
        y, eye, (((0,), (0,)), ((), ())),
        preferred_element_type=jnp.float32)


_repack = pl.pallas_call(
    _repack_body,
    grid=(_RP_GRID,),
    in_specs=[pl.BlockSpec((EMB_DIM, _RP_W), lambda j: (0, j))],
    out_specs=pl.BlockSpec((_RP_W // 2, 128), lambda j: (j, 0)),
    out_shape=jax.ShapeDtypeStruct((_N2, 128), jnp.float32),
)


def kernel(pos_u, pos_v, neg_v, u_weight, v_weight):
    pos_u = pos_u.astype(jnp.int32)
    pos_v = pos_v.astype(jnp.int32)
    neg_t = neg_v.astype(jnp.int32).T

    u2 = _repack(u_weight.T)
    v2 = _repack(v_weight.T)

    pos_dots, neg_dots = _sc_dots(pos_u, pos_v, neg_t, u2, v2)

    out = pl.pallas_call(
        _reduce_body,
        out_shape=jax.ShapeDtypeStruct((1, 1), jnp.float32),
    )(pos_dots.reshape(BATCH // 128, 128),
      neg_dots.reshape(BATCH * NEG // 128, 128))
    return out[0, 0]
